# Initial kernel scaffold; baseline (speedup 1.0000x reference)
import jax
import jax.numpy as jnp
from jax.experimental import pallas as pl


def kernel(x, edge_index, W1, b1, W2, b2):
    raise NotImplementedError("write your pallas kernel here")


# speedup vs baseline: 27.5056x; 27.5056x over previous
import functools

import jax
import jax.numpy as jnp
from jax import lax
from jax.experimental import pallas as pl
from jax.experimental.pallas import tpu as pltpu
from jax.experimental.pallas import tpu_sc as plsc

N = 10000
E = 320000
D_IN = 128
HIDDEN = 16
NUM_CLASSES = 40

NC = 2
NS = 16
NW = NC * NS
CHUNK = 128
CH = 80
E_PAD = NW * CH * CHUNK
N_PAD = 10240
ROWS_PER_TILE = N_PAD // NS

_mesh = plsc.VectorSubcoreMesh(core_axis_name="c", subcore_axis_name="s")
_sc_params = pltpu.CompilerParams(use_tc_tiling_on_sc=False)


def _zero_vmem(buf, d):
    def body(r, _):
        for l in range(d // 16):
            buf[r, pl.ds(16 * l, 16)] = jnp.zeros((16,), jnp.float32)
        return 0
    lax.fori_loop(0, 128, body, 0)


def _fill_ones(buf, d):
    def body(r, _):
        for l in range(d // 16):
            buf[r, pl.ds(16 * l, 16)] = jnp.ones((16,), jnp.float32)
        return 0
    lax.fori_loop(0, 128, body, 0)


def _make_degree_kernel():
    d = 16

    @functools.partial(
        pl.kernel,
        out_type=jax.ShapeDtypeStruct((NC, N_PAD, d), jnp.float32),
        mesh=_mesh,
        compiler_params=_sc_params,
        scratch_types=[
            pltpu.VMEM((CH, CHUNK), jnp.int32),
            pltpu.VMEM((CHUNK, d), jnp.float32),
            pltpu.VMEM((CHUNK, d), jnp.float32),
            pltpu.VMEM_SHARED((N_PAD, d), jnp.float32),
        ],
    )
    def deg_kernel(col_hbm, out_hbm, col_v, ones_v, zero_v, acc):
        c = lax.axis_index("c")
        s = lax.axis_index("s")
        w = c * NS + s
        pltpu.sync_copy(col_hbm.at[w], col_v)
        _fill_ones(ones_v, d)
        _zero_vmem(zero_v, d)
        for i in range(ROWS_PER_TILE // 128):
            pltpu.sync_copy(zero_v, acc.at[pl.ds(s * ROWS_PER_TILE + i * 128, 128)])
        plsc.subcore_barrier()

        def step(j, _):
            pltpu.sync_copy(ones_v, acc.at[col_v.at[j]], add=True)
            return 0
        lax.fori_loop(0, CH, step, 0)
        plsc.subcore_barrier()
        sl = pl.ds(s * ROWS_PER_TILE, ROWS_PER_TILE)
        pltpu.sync_copy(acc.at[sl], out_hbm.at[c, sl])

    return deg_kernel


def _make_agg_kernel(d):
    @functools.partial(
        pl.kernel,
        out_type=jax.ShapeDtypeStruct((NC, N_PAD, d), jnp.float32),
        mesh=_mesh,
        compiler_params=_sc_params,
        scratch_types=[
            pltpu.VMEM((CH, CHUNK), jnp.int32),
            pltpu.VMEM((CH, CHUNK), jnp.int32),
            pltpu.VMEM((CHUNK, d), jnp.float32),
            pltpu.VMEM((CHUNK, d), jnp.float32),
            pltpu.VMEM((128, d), jnp.float32),
            pltpu.VMEM_SHARED((N_PAD, d), jnp.float32),
            pltpu.SemaphoreType.DMA,
            pltpu.SemaphoreType.DMA,
        ],
    )
    def agg_kernel(table_hbm, row_hbm, col_hbm, out_hbm,
                   row_v, col_v, rows_a, rows_b, zero_v, acc, sem_a, sem_b):
        c = lax.axis_index("c")
        s = lax.axis_index("s")
        w = c * NS + s
        pltpu.sync_copy(row_hbm.at[w], row_v)
        pltpu.sync_copy(col_hbm.at[w], col_v)
        _zero_vmem(zero_v, d)
        for i in range(ROWS_PER_TILE // 128):
            pltpu.sync_copy(zero_v, acc.at[pl.ds(s * ROWS_PER_TILE + i * 128, 128)])
        plsc.subcore_barrier()

        pltpu.async_copy(table_hbm.at[row_v.at[0]], rows_a, sem_a)
        pltpu.async_copy(table_hbm.at[row_v.at[1]], rows_b, sem_b)

        def step(j0, _):
            pltpu.make_async_copy(table_hbm.at[row_v.at[j0]], rows_a, sem_a).wait()
            pltpu.sync_copy(rows_a, acc.at[col_v.at[j0]], add=True)

            @pl.when(j0 + 2 < CH)
            def _():
                pltpu.async_copy(table_hbm.at[row_v.at[j0 + 2]], rows_a, sem_a)

            pltpu.make_async_copy(table_hbm.at[row_v.at[j0 + 1]], rows_b, sem_b).wait()
            pltpu.sync_copy(rows_b, acc.at[col_v.at[j0 + 1]], add=True)

            @pl.when(j0 + 3 < CH)
            def _():
                pltpu.async_copy(table_hbm.at[row_v.at[j0 + 3]], rows_b, sem_b)
            return 0

        lax.fori_loop(0, CH // 2, lambda i, _: step(2 * i, _), 0)
        plsc.subcore_barrier()
        sl = pl.ds(s * ROWS_PER_TILE, ROWS_PER_TILE)
        pltpu.sync_copy(acc.at[sl], out_hbm.at[c, sl])

    return agg_kernel


_deg_kernel = _make_degree_kernel()
_agg16 = _make_agg_kernel(HIDDEN)
_agg48 = _make_agg_kernel(48)


def _tc1_body(x_ref, w1_ref, degp_ref, g1_ref, dinv_ref):
    deg = degp_ref[0, :, :8] + degp_ref[1, :, :8] + 1.0
    dinv = lax.rsqrt(deg)
    h = jnp.dot(x_ref[...], w1_ref[...], preferred_element_type=jnp.float32)
    g1_ref[...] = h * dinv[:, :1]
    dinv_ref[...] = dinv


def _tc1(x_p, W1, degp):
    b = 512
    grid = N_PAD // b
    return pl.pallas_call(
        _tc1_body,
        grid=(grid,),
        in_specs=[
            pl.BlockSpec((b, D_IN), lambda j: (j, 0)),
            pl.BlockSpec((D_IN, HIDDEN), lambda j: (0, 0)),
            pl.BlockSpec((NC, b, 16), lambda j: (0, j, 0)),
        ],
        out_specs=[
            pl.BlockSpec((b, HIDDEN), lambda j: (j, 0)),
            pl.BlockSpec((b, 8), lambda j: (j, 0)),
        ],
        out_shape=[
            jax.ShapeDtypeStruct((N_PAD, HIDDEN), jnp.float32),
            jax.ShapeDtypeStruct((N_PAD, 8), jnp.float32),
        ],
    )(x_p, W1, degp)


def _tc2_body(s1p_ref, g1_ref, dinv_ref, b1_ref, w2p_ref, g2_ref):
    agg = s1p_ref[0] + s1p_ref[1] + g1_ref[...]
    dinv = dinv_ref[:, :1]
    h = jnp.maximum(agg * dinv + b1_ref[...], 0.0)
    g2_ref[...] = jnp.dot(h, w2p_ref[...], preferred_element_type=jnp.float32) * dinv


def _tc2(s1p, g1, dinv, b1, W2p):
    b = 512
    grid = N_PAD // b
    return pl.pallas_call(
        _tc2_body,
        grid=(grid,),
        in_specs=[
            pl.BlockSpec((NC, b, HIDDEN), lambda j: (0, j, 0)),
            pl.BlockSpec((b, HIDDEN), lambda j: (j, 0)),
            pl.BlockSpec((b, 8), lambda j: (j, 0)),
            pl.BlockSpec((1, HIDDEN), lambda j: (0, 0)),
            pl.BlockSpec((HIDDEN, 48), lambda j: (0, 0)),
        ],
        out_specs=pl.BlockSpec((b, 48), lambda j: (j, 0)),
        out_shape=jax.ShapeDtypeStruct((N_PAD, 48), jnp.float32),
    )(s1p, g1, dinv, b1, W2p)


def _tc3_body(s2p_ref, g2_ref, dinv_ref, b2_ref, out_ref):
    agg = s2p_ref[0] + s2p_ref[1] + g2_ref[...]
    t = (agg * dinv_ref[:, :1])[:, :NUM_CLASSES] + b2_ref[...]
    m = jnp.max(t, axis=1, keepdims=True)
    lse = jnp.log(jnp.sum(jnp.exp(t - m), axis=1, keepdims=True))
    out_ref[...] = t - m - lse


def _tc3(s2p, g2, dinv, b2):
    b = 1000
    grid = N // b
    return pl.pallas_call(
        _tc3_body,
        grid=(grid,),
        in_specs=[
            pl.BlockSpec((NC, b, 48), lambda j: (0, j, 0)),
            pl.BlockSpec((b, 48), lambda j: (j, 0)),
            pl.BlockSpec((b, 8), lambda j: (j, 0)),
            pl.BlockSpec((1, NUM_CLASSES), lambda j: (0, 0)),
        ],
        out_specs=pl.BlockSpec((b, NUM_CLASSES), lambda j: (j, 0)),
        out_shape=jax.ShapeDtypeStruct((N, NUM_CLASSES), jnp.float32),
    )(s2p, g2, dinv, b2)


def kernel(x, edge_index, W1, b1, W2, b2):
    row = edge_index[0]
    col = edge_index[1]
    pad = E_PAD - E
    row_r = jnp.concatenate([row, jnp.zeros((pad,), jnp.int32)]).reshape(NW, CH, CHUNK)
    col_r = jnp.concatenate([col, jnp.full((pad,), N, jnp.int32)]).reshape(NW, CH, CHUNK)

    degp = _deg_kernel(col_r)
    x_p = jnp.pad(x, ((0, N_PAD - N), (0, 0)))
    g1, dinv = _tc1(x_p, W1, degp)
    s1p = _agg16(g1, row_r, col_r)
    W2p = jnp.pad(W2, ((0, 0), (0, 48 - NUM_CLASSES)))
    g2 = _tc2(s1p, g1, dinv, b1.reshape(1, HIDDEN), W2p)
    s2p = _agg48(g2, row_r, col_r)
    return _tc3(s2p, g2, dinv, b2.reshape(1, NUM_CLASSES))


# speedup vs baseline: 35.0231x; 1.2733x over previous
import functools

import jax
import jax.numpy as jnp
from jax import lax
from jax.experimental import pallas as pl
from jax.experimental.pallas import tpu as pltpu
from jax.experimental.pallas import tpu_sc as plsc

N = 10000
E = 320000
D_IN = 128
HIDDEN = 16
NUM_CLASSES = 40

NC = 2
NS = 16
NW = NC * NS
CHUNK = 128
CH = 80
E_PAD = NW * CH * CHUNK
N_PAD = 10240
ROWS_PER_TILE = N_PAD // NS
EPT = CH * CHUNK

_mesh = plsc.VectorSubcoreMesh(core_axis_name="c", subcore_axis_name="s")
_sc_params = pltpu.CompilerParams(use_tc_tiling_on_sc=False)
_sc_params_reg = pltpu.CompilerParams(use_tc_tiling_on_sc=False,
                                      needs_layout_passes=False)


def _zero_vmem(buf, d):
    def body(r, _):
        for l in range(d // 16):
            buf[r, pl.ds(16 * l, 16)] = jnp.zeros((16,), jnp.float32)
        return 0
    lax.fori_loop(0, 128, body, 0)


def _make_degree_kernel():
    @functools.partial(
        pl.kernel,
        out_type=jax.ShapeDtypeStruct((NW, N_PAD), jnp.float32),
        mesh=_mesh,
        compiler_params=_sc_params_reg,
        scratch_types=[
            pltpu.VMEM((EPT // 16, 16), jnp.int32),
            pltpu.VMEM((N_PAD,), jnp.float32),
        ],
    )
    def deg_kernel(col_hbm, out_hbm, col_v, hist):
        c = lax.axis_index("c")
        s = lax.axis_index("s")
        w = c * NS + s
        pltpu.sync_copy(col_hbm.at[w], col_v)

        def zero(i, _):
            hist[pl.ds(i * 16, 16)] = jnp.zeros((16,), jnp.float32)
            return 0
        lax.fori_loop(0, N_PAD // 16, zero, 0)

        ones = jnp.ones((16,), jnp.float32)

        def step(j, _):
            plsc.addupdate_scatter(hist, [col_v[j]], ones)
            return 0
        lax.fori_loop(0, EPT // 16, step, 0)
        pltpu.sync_copy(hist, out_hbm.at[w])

    return deg_kernel


def _make_agg_kernel(d):
    @functools.partial(
        pl.kernel,
        out_type=jax.ShapeDtypeStruct((NC, N_PAD, d), jnp.float32),
        mesh=_mesh,
        compiler_params=_sc_params,
        scratch_types=[
            pltpu.VMEM((CH, CHUNK), jnp.int32),
            pltpu.VMEM((CH, CHUNK), jnp.int32),
            pltpu.VMEM((CHUNK, d), jnp.float32),
            pltpu.VMEM((CHUNK, d), jnp.float32),
            pltpu.VMEM((128, d), jnp.float32),
            pltpu.VMEM_SHARED((N_PAD, d), jnp.float32),
            pltpu.SemaphoreType.DMA,
            pltpu.SemaphoreType.DMA,
        ],
    )
    def agg_kernel(table_hbm, row_hbm, col_hbm, out_hbm,
                   row_v, col_v, rows_a, rows_b, zero_v, acc, sem_a, sem_b):
        c = lax.axis_index("c")
        s = lax.axis_index("s")
        w = c * NS + s
        pltpu.sync_copy(row_hbm.at[w], row_v)
        pltpu.sync_copy(col_hbm.at[w], col_v)
        _zero_vmem(zero_v, d)
        for i in range(ROWS_PER_TILE // 128):
            pltpu.sync_copy(zero_v, acc.at[pl.ds(s * ROWS_PER_TILE + i * 128, 128)])
        plsc.subcore_barrier()

        pltpu.async_copy(table_hbm.at[row_v.at[0]], rows_a, sem_a)
        pltpu.async_copy(table_hbm.at[row_v.at[1]], rows_b, sem_b)

        def step(j0, _):
            pltpu.make_async_copy(table_hbm.at[row_v.at[j0]], rows_a, sem_a).wait()
            pltpu.sync_copy(rows_a, acc.at[col_v.at[j0]], add=True)

            @pl.when(j0 + 2 < CH)
            def _():
                pltpu.async_copy(table_hbm.at[row_v.at[j0 + 2]], rows_a, sem_a)

            pltpu.make_async_copy(table_hbm.at[row_v.at[j0 + 1]], rows_b, sem_b).wait()
            pltpu.sync_copy(rows_b, acc.at[col_v.at[j0 + 1]], add=True)

            @pl.when(j0 + 3 < CH)
            def _():
                pltpu.async_copy(table_hbm.at[row_v.at[j0 + 3]], rows_b, sem_b)
            return 0

        lax.fori_loop(0, CH // 2, lambda i, _: step(2 * i, _), 0)
        plsc.subcore_barrier()
        sl = pl.ds(s * ROWS_PER_TILE, ROWS_PER_TILE)
        pltpu.sync_copy(acc.at[sl], out_hbm.at[c, sl])

    return agg_kernel


_deg_kernel = _make_degree_kernel()
_agg16 = _make_agg_kernel(HIDDEN)


def _tc1_body(x_ref, w1_ref, degp_ref, g1_ref, dinv_ref):
    deg = jnp.sum(degp_ref[...], axis=0) + 1.0
    dinv = lax.rsqrt(deg)[:, None]
    h = jnp.dot(x_ref[...], w1_ref[...], preferred_element_type=jnp.float32)
    g1_ref[...] = h * dinv
    dinv_ref[...] = jnp.broadcast_to(dinv, dinv_ref.shape)


def _tc1(x_p, W1, degp):
    b = 512
    grid = N_PAD // b
    return pl.pallas_call(
        _tc1_body,
        grid=(grid,),
        in_specs=[
            pl.BlockSpec((b, D_IN), lambda j: (j, 0)),
            pl.BlockSpec((D_IN, HIDDEN), lambda j: (0, 0)),
            pl.BlockSpec((NW, b), lambda j: (0, j)),
        ],
        out_specs=[
            pl.BlockSpec((b, HIDDEN), lambda j: (j, 0)),
            pl.BlockSpec((b, 8), lambda j: (j, 0)),
        ],
        out_shape=[
            jax.ShapeDtypeStruct((N_PAD, HIDDEN), jnp.float32),
            jax.ShapeDtypeStruct((N_PAD, 8), jnp.float32),
        ],
    )(x_p, W1, degp)


def _tc2_body(s1p_ref, g1_ref, dinv_ref, b1_ref, u_ref):
    agg = s1p_ref[0] + s1p_ref[1] + g1_ref[...]
    dinv = dinv_ref[:, :1]
    h = jnp.maximum(agg * dinv + b1_ref[...], 0.0)
    u_ref[...] = h * dinv


def _tc2(s1p, g1, dinv, b1):
    b = 512
    grid = N_PAD // b
    return pl.pallas_call(
        _tc2_body,
        grid=(grid,),
        in_specs=[
            pl.BlockSpec((NC, b, HIDDEN), lambda j: (0, j, 0)),
            pl.BlockSpec((b, HIDDEN), lambda j: (j, 0)),
            pl.BlockSpec((b, 8), lambda j: (j, 0)),
            pl.BlockSpec((1, HIDDEN), lambda j: (0, 0)),
        ],
        out_specs=pl.BlockSpec((b, HIDDEN), lambda j: (j, 0)),
        out_shape=jax.ShapeDtypeStruct((N_PAD, HIDDEN), jnp.float32),
    )(s1p, g1, dinv, b1)


def _tc3_body(s2p_ref, u_ref, dinv_ref, w2_ref, b2_ref, out_ref):
    agg = (s2p_ref[0] + s2p_ref[1] + u_ref[...]) * dinv_ref[:, :1]
    t = jnp.dot(agg, w2_ref[...], preferred_element_type=jnp.float32) + b2_ref[...]
    m = jnp.max(t, axis=1, keepdims=True)
    lse = jnp.log(jnp.sum(jnp.exp(t - m), axis=1, keepdims=True))
    out_ref[...] = t - m - lse


def _tc3(s2p, u, dinv, W2, b2):
    b = 1000
    grid = N // b
    return pl.pallas_call(
        _tc3_body,
        grid=(grid,),
        in_specs=[
            pl.BlockSpec((NC, b, HIDDEN), lambda j: (0, j, 0)),
            pl.BlockSpec((b, HIDDEN), lambda j: (j, 0)),
            pl.BlockSpec((b, 8), lambda j: (j, 0)),
            pl.BlockSpec((HIDDEN, NUM_CLASSES), lambda j: (0, 0)),
            pl.BlockSpec((1, NUM_CLASSES), lambda j: (0, 0)),
        ],
        out_specs=pl.BlockSpec((b, NUM_CLASSES), lambda j: (j, 0)),
        out_shape=jax.ShapeDtypeStruct((N, NUM_CLASSES), jnp.float32),
    )(s2p, u, dinv, W2, b2)


def kernel(x, edge_index, W1, b1, W2, b2):
    row = edge_index[0]
    col = edge_index[1]
    pad = E_PAD - E
    row_r = jnp.concatenate([row, jnp.zeros((pad,), jnp.int32)]).reshape(NW, CH, CHUNK)
    col_r = jnp.concatenate([col, jnp.full((pad,), N, jnp.int32)]).reshape(NW, CH, CHUNK)

    col_d = col_r.reshape(NW, EPT // 16, 16)
    degp = _deg_kernel(col_d)
    x_p = jnp.pad(x, ((0, N_PAD - N), (0, 0)))
    g1, dinv = _tc1(x_p, W1, degp)
    s1p = _agg16(g1, row_r, col_r)
    u = _tc2(s1p, g1, dinv, b1.reshape(1, HIDDEN))
    s2p = _agg16(u, row_r, col_r)
    return _tc3(s2p, u, dinv, W2, b2.reshape(1, NUM_CLASSES))


# speedup vs baseline: 50.4168x; 1.4395x over previous
import functools

import jax
import jax.numpy as jnp
from jax import lax
from jax.experimental import pallas as pl
from jax.experimental.pallas import tpu as pltpu
from jax.experimental.pallas import tpu_sc as plsc

N = 10000
E = 320000
D_IN = 128
HIDDEN = 16
NUM_CLASSES = 40

NC = 2
NS = 16
NW = NC * NS
CHUNK = 125
CH = 80
N_PAD = 10240
ROWS_PER_TILE = N_PAD // NS
EPT = CH * CHUNK

_mesh = plsc.VectorSubcoreMesh(core_axis_name="c", subcore_axis_name="s")
_sc_params = pltpu.CompilerParams(use_tc_tiling_on_sc=False)
_sc_params_reg = pltpu.CompilerParams(use_tc_tiling_on_sc=False,
                                      needs_layout_passes=False)


def _zero_vmem(buf, d):
    def body(r, _):
        for l in range(d // 16):
            buf[r, pl.ds(16 * l, 16)] = jnp.zeros((16,), jnp.float32)
        return 0
    lax.fori_loop(0, 128, body, 0)


def _make_degree_kernel():
    @functools.partial(
        pl.kernel,
        out_type=jax.ShapeDtypeStruct((NW, N_PAD), jnp.float32),
        mesh=_mesh,
        compiler_params=_sc_params_reg,
        scratch_types=[
            pltpu.VMEM((EPT // 16, 16), jnp.int32),
            pltpu.VMEM((N_PAD,), jnp.float32),
        ],
    )
    def deg_kernel(col_hbm, out_hbm, col_v, hist):
        c = lax.axis_index("c")
        s = lax.axis_index("s")
        w = c * NS + s
        pltpu.sync_copy(col_hbm.at[w], col_v)

        def zero(i, _):
            hist[pl.ds(i * 16, 16)] = jnp.zeros((16,), jnp.float32)
            return 0
        lax.fori_loop(0, N_PAD // 16, zero, 0)

        ones = jnp.ones((16,), jnp.float32)

        def step(j, _):
            plsc.addupdate_scatter(hist, [col_v[j]], ones)
            return 0
        lax.fori_loop(0, EPT // 16, step, 0)
        pltpu.sync_copy(hist, out_hbm.at[w])

    return deg_kernel


def _make_agg_kernel(d):
    @functools.partial(
        pl.kernel,
        out_type=jax.ShapeDtypeStruct((NC, N_PAD, d), jnp.float32),
        mesh=_mesh,
        compiler_params=_sc_params,
        scratch_types=[
            pltpu.VMEM((CH, CHUNK), jnp.int32),
            pltpu.VMEM((CH, CHUNK), jnp.int32),
            pltpu.VMEM((CHUNK, d), jnp.float32),
            pltpu.VMEM((CHUNK, d), jnp.float32),
            pltpu.VMEM((128, d), jnp.float32),
            pltpu.VMEM_SHARED((N_PAD, d), jnp.float32),
            pltpu.SemaphoreType.DMA,
            pltpu.SemaphoreType.DMA,
        ],
    )
    def agg_kernel(table_hbm, row_hbm, col_hbm, out_hbm,
                   row_v, col_v, rows_a, rows_b, zero_v, acc, sem_a, sem_b):
        c = lax.axis_index("c")
        s = lax.axis_index("s")
        w = c * NS + s
        pltpu.sync_copy(row_hbm.at[w], row_v)
        pltpu.sync_copy(col_hbm.at[w], col_v)
        _zero_vmem(zero_v, d)
        for i in range(ROWS_PER_TILE // 128):
            pltpu.sync_copy(zero_v, acc.at[pl.ds(s * ROWS_PER_TILE + i * 128, 128)])
        plsc.subcore_barrier()

        pltpu.async_copy(table_hbm.at[row_v.at[0]], rows_a, sem_a)
        pltpu.async_copy(table_hbm.at[row_v.at[1]], rows_b, sem_b)

        def step(j0, _):
            pltpu.make_async_copy(table_hbm.at[row_v.at[j0]], rows_a, sem_a).wait()
            pltpu.sync_copy(rows_a, acc.at[col_v.at[j0]], add=True)

            @pl.when(j0 + 2 < CH)
            def _():
                pltpu.async_copy(table_hbm.at[row_v.at[j0 + 2]], rows_a, sem_a)

            pltpu.make_async_copy(table_hbm.at[row_v.at[j0 + 1]], rows_b, sem_b).wait()
            pltpu.sync_copy(rows_b, acc.at[col_v.at[j0 + 1]], add=True)

            @pl.when(j0 + 3 < CH)
            def _():
                pltpu.async_copy(table_hbm.at[row_v.at[j0 + 3]], rows_b, sem_b)
            return 0

        lax.fori_loop(0, CH // 2, lambda i, _: step(2 * i, _), 0)
        plsc.subcore_barrier()
        sl = pl.ds(s * ROWS_PER_TILE, ROWS_PER_TILE)
        pltpu.sync_copy(acc.at[sl], out_hbm.at[c, sl])

    return agg_kernel


_deg_kernel = _make_degree_kernel()
_agg16 = _make_agg_kernel(HIDDEN)


def _tc1_body(x_ref, w1_ref, degp_ref, g1_ref, dinv_ref):
    deg = jnp.sum(degp_ref[...], axis=0)[:N, None] + 1.0
    dinv = lax.rsqrt(deg)
    h = jnp.dot(x_ref[...], w1_ref[...], preferred_element_type=jnp.float32)
    g1_ref[...] = h * dinv
    dinv_ref[...] = jnp.broadcast_to(dinv, dinv_ref.shape)


def _tc1(x, W1, degp):
    return pl.pallas_call(
        _tc1_body,
        out_shape=[
            jax.ShapeDtypeStruct((N, HIDDEN), jnp.float32),
            jax.ShapeDtypeStruct((N, 8), jnp.float32),
        ],
    )(x, W1, degp)


def _tc2_body(s1p_ref, g1_ref, dinv_ref, b1_ref, u_ref):
    agg = s1p_ref[0, :N] + s1p_ref[1, :N] + g1_ref[...]
    dinv = dinv_ref[:, :1]
    h = jnp.maximum(agg * dinv + b1_ref[...], 0.0)
    u_ref[...] = h * dinv


def _tc2(s1p, g1, dinv, b1):
    return pl.pallas_call(
        _tc2_body,
        out_shape=jax.ShapeDtypeStruct((N, HIDDEN), jnp.float32),
    )(s1p, g1, dinv, b1)


def _tc3_body(s2p_ref, u_ref, dinv_ref, w2_ref, b2_ref, out_ref):
    agg = (s2p_ref[0, :N] + s2p_ref[1, :N] + u_ref[...]) * dinv_ref[:, :1]
    t = jnp.dot(agg, w2_ref[...], preferred_element_type=jnp.float32) + b2_ref[...]
    m = jnp.max(t, axis=1, keepdims=True)
    lse = jnp.log(jnp.sum(jnp.exp(t - m), axis=1, keepdims=True))
    out_ref[...] = t - m - lse


def _tc3(s2p, u, dinv, W2, b2):
    return pl.pallas_call(
        _tc3_body,
        out_shape=jax.ShapeDtypeStruct((N, NUM_CLASSES), jnp.float32),
    )(s2p, u, dinv, W2, b2)


def kernel(x, edge_index, W1, b1, W2, b2):
    row_r = edge_index[0].reshape(NW, CH, CHUNK)
    col_r = edge_index[1].reshape(NW, CH, CHUNK)
    col_d = edge_index[1].reshape(NW, EPT // 16, 16)

    degp = _deg_kernel(col_d)
    g1, dinv = _tc1(x, W1, degp)
    s1p = _agg16(g1, row_r, col_r)
    u = _tc2(s1p, g1, dinv, b1)
    s2p = _agg16(u, row_r, col_r)
    return _tc3(s2p, u, dinv, W2, b2)
